# baseline (device time: 256839 ns/iter reference)
import numpy as np
import jax
import jax.numpy as jnp
from jax import lax
from jax.experimental import pallas as pl
from jax.experimental.pallas import tpu as pltpu

N_DEV = 32
B = 2
SQ = 512
SKV = 512
HQ_G = 256
DH = 64
H_LOC = HQ_G // N_DEV
CH = SQ // N_DEV
D_MODEL = 768


def _body(x_ref, wq_ref, k_hbm, v_hbm, wo_ref, out_ref,
          k_vmem, v_vmem, send_buf, recv_buf, red_buf,
          k_sems, v_sems,
          p1_send_sems, p1_recv_sems, p2_send_sems, p2_recv_sems):
    me = lax.axis_index("i")

    def kv_copy(hbm, vmem, sems, b):
        return pltpu.make_async_copy(
            hbm.at[b, :, pl.ds(me * H_LOC, H_LOC), :], vmem.at[b], sems.at[b]
        )

    for b in range(B):
        kv_copy(k_hbm, k_vmem, k_sems, b).start()
        kv_copy(v_hbm, v_vmem, v_sems, b).start()

    barrier = pltpu.get_barrier_semaphore()
    for nbr in range(N_DEV):
        @pl.when(nbr != me)
        def _(nbr=nbr):
            pl.semaphore_signal(
                barrier, inc=1,
                device_id=(nbr,), device_id_type=pl.DeviceIdType.MESH,
            )
    pl.semaphore_wait(barrier, N_DEV - 1)

    xb = x_ref[...].reshape(B * SQ, D_MODEL).astype(jnp.bfloat16)
    q = lax.dot_general(
        xb, wq_ref[...].astype(jnp.bfloat16), (((1,), (0,)), ((), ())),
        preferred_element_type=jnp.float32,
    )

    qb = lax.broadcasted_iota(jnp.int32, (SQ, SKV), 0) // 64
    kb = lax.broadcasted_iota(jnp.int32, (SQ, SKV), 1) // 64
    keep = (qb == kb) | ((kb % 4) == (qb % 4))
    mask_add = jnp.where(keep, 0.0, -1e9).astype(jnp.float32)

    for b in range(B):
        kv_copy(k_hbm, k_vmem, k_sems, b).wait()
        kv_copy(v_hbm, v_vmem, v_sems, b).wait()

    ctx_rows = []
    for b in range(B):
        head_cols = []
        for h in range(H_LOC):
            qbh = q[b * SQ:(b + 1) * SQ, h * DH:(h + 1) * DH]
            qbh = qbh.astype(jnp.bfloat16)
            kbh = k_vmem[b, :, h, :].astype(jnp.bfloat16)
            vbh = v_vmem[b, :, h, :].astype(jnp.bfloat16)
            s = lax.dot_general(
                qbh, kbh, (((1,), (1,)), ((), ())),
                preferred_element_type=jnp.float32,
            )
            s = s * 0.125 + mask_add
            m = jnp.max(s, axis=1, keepdims=True)
            w = jnp.exp(s - m)
            w = w / jnp.sum(w, axis=1, keepdims=True)
            cbh = lax.dot_general(
                w.astype(jnp.bfloat16), vbh, (((1,), (0,)), ((), ())),
                preferred_element_type=jnp.float32,
            )
            head_cols.append(cbh)
        ctx_rows.append(jnp.concatenate(head_cols, axis=1))
    ctx = jnp.concatenate(ctx_rows, axis=0)

    partial = lax.dot_general(
        ctx.astype(jnp.bfloat16), wo_ref[...].astype(jnp.bfloat16),
        (((1,), (0,)), ((), ())),
        preferred_element_type=jnp.float32,
    ).reshape(B, SQ, D_MODEL)

    for j in range(N_DEV):
        send_buf[j] = partial[:, j * CH:(j + 1) * CH, :].astype(jnp.bfloat16)

    for j in range(N_DEV):
        @pl.when(j != me)
        def _(j=j):
            rdma = pltpu.make_async_remote_copy(
                src_ref=send_buf.at[j],
                dst_ref=recv_buf.at[me],
                send_sem=p1_send_sems.at[j],
                recv_sem=p1_recv_sems.at[me],
                device_id=(j,), device_id_type=pl.DeviceIdType.MESH,
            )
            rdma.start()

    recv_buf[me] = send_buf[me]

    for s in range(N_DEV):
        @pl.when(s != me)
        def _(s=s):
            rdma = pltpu.make_async_remote_copy(
                src_ref=send_buf.at[s],
                dst_ref=recv_buf.at[s],
                send_sem=p1_send_sems.at[s],
                recv_sem=p1_recv_sems.at[s],
                device_id=(s,), device_id_type=pl.DeviceIdType.MESH,
            )
            rdma.wait_recv()

    acc = [recv_buf[s].astype(jnp.float32) for s in range(N_DEV)]
    while len(acc) > 1:
        acc = [acc[i] + acc[i + 1] for i in range(0, len(acc) - 1, 2)] + (
            [acc[-1]] if len(acc) % 2 else []
        )
    red = acc[0].astype(jnp.bfloat16)
    red_buf[...] = red
    out_ref[:, pl.ds(me * CH, CH), :] = red

    for j in range(N_DEV):
        @pl.when(j != me)
        def _(j=j):
            rdma = pltpu.make_async_remote_copy(
                src_ref=red_buf,
                dst_ref=out_ref.at[:, pl.ds(me * CH, CH), :],
                send_sem=p2_send_sems.at[j],
                recv_sem=p2_recv_sems.at[me],
                device_id=(j,), device_id_type=pl.DeviceIdType.MESH,
            )
            rdma.start()

    for s in range(N_DEV):
        @pl.when(s != me)
        def _(s=s):
            rdma = pltpu.make_async_remote_copy(
                src_ref=red_buf,
                dst_ref=out_ref.at[:, pl.ds(s * CH, CH), :],
                send_sem=p2_send_sems.at[s],
                recv_sem=p2_recv_sems.at[s],
                device_id=(s,), device_id_type=pl.DeviceIdType.MESH,
            )
            rdma.wait_recv()

    for j in range(N_DEV):
        @pl.when(j != me)
        def _(j=j):
            s1 = pltpu.make_async_remote_copy(
                src_ref=send_buf.at[j],
                dst_ref=recv_buf.at[j],
                send_sem=p1_send_sems.at[j],
                recv_sem=p1_recv_sems.at[j],
                device_id=(j,), device_id_type=pl.DeviceIdType.MESH,
            )
            s1.wait_send()
            s2 = pltpu.make_async_remote_copy(
                src_ref=red_buf,
                dst_ref=out_ref.at[:, pl.ds(j * CH, CH), :],
                send_sem=p2_send_sems.at[j],
                recv_sem=p2_recv_sems.at[j],
                device_id=(j,), device_id_type=pl.DeviceIdType.MESH,
            )
            s2.wait_send()


def kernel(x, Wq, K_ext, V_ext, Wo):
    return pl.pallas_call(
        _body,
        out_shape=jax.ShapeDtypeStruct((B, SQ, D_MODEL), jnp.bfloat16),
        in_specs=[
            pl.BlockSpec(memory_space=pltpu.VMEM),
            pl.BlockSpec(memory_space=pltpu.VMEM),
            pl.BlockSpec(memory_space=pl.ANY),
            pl.BlockSpec(memory_space=pl.ANY),
            pl.BlockSpec(memory_space=pltpu.VMEM),
        ],
        out_specs=pl.BlockSpec(memory_space=pltpu.VMEM),
        scratch_shapes=[
            pltpu.VMEM((B, SKV, H_LOC, DH), jnp.float32),
            pltpu.VMEM((B, SKV, H_LOC, DH), jnp.float32),
            pltpu.VMEM((N_DEV, B, CH, D_MODEL), jnp.bfloat16),
            pltpu.VMEM((N_DEV, B, CH, D_MODEL), jnp.bfloat16),
            pltpu.VMEM((B, CH, D_MODEL), jnp.bfloat16),
            pltpu.SemaphoreType.DMA((B,)),
            pltpu.SemaphoreType.DMA((B,)),
            pltpu.SemaphoreType.DMA((N_DEV,)),
            pltpu.SemaphoreType.DMA((N_DEV,)),
            pltpu.SemaphoreType.DMA((N_DEV,)),
            pltpu.SemaphoreType.DMA((N_DEV,)),
        ],
        compiler_params=pltpu.CompilerParams(collective_id=0),
    )(x, Wq, K_ext, V_ext, Wo)


# device time: 128968 ns/iter; 1.9915x vs baseline; 1.9915x over previous
import numpy as np
import jax
import jax.numpy as jnp
from jax import lax
from jax.experimental import pallas as pl
from jax.experimental.pallas import tpu as pltpu

N_DEV = 32
B = 2
SQ = 512
SKV = 512
HQ_G = 256
DH = 64
H_LOC = HQ_G // N_DEV
CH = SQ // N_DEV
D_MODEL = 768


def _body(x_ref, wq_ref, k_ref, v_ref, wo_ref, out_ref,
          send_buf, recv_buf, red_buf,
          p1_send_sems, p1_recv_sems, p2_send_sems, p2_recv_sems):
    me = lax.axis_index("i")

    barrier = pltpu.get_barrier_semaphore()
    for nbr in range(N_DEV):
        @pl.when(nbr != me)
        def _(nbr=nbr):
            pl.semaphore_signal(
                barrier, inc=1,
                device_id=(nbr,), device_id_type=pl.DeviceIdType.MESH,
            )
    pl.semaphore_wait(barrier, N_DEV - 1)

    xb = x_ref[...].reshape(B * SQ, D_MODEL).astype(jnp.bfloat16)
    q = lax.dot_general(
        xb, wq_ref[...], (((1,), (0,)), ((), ())),
        preferred_element_type=jnp.float32,
    )

    qb = lax.broadcasted_iota(jnp.int32, (SQ, SKV), 0) // 64
    kb = lax.broadcasted_iota(jnp.int32, (SQ, SKV), 1) // 64
    keep = (qb == kb) | ((kb % 4) == (qb % 4))
    mask_add = jnp.where(keep, 0.0, -1e9).astype(jnp.float32)

    ctx_rows = []
    for b in range(B):
        head_cols = []
        for h in range(H_LOC):
            qbh = q[b * SQ:(b + 1) * SQ, h * DH:(h + 1) * DH]
            qbh = qbh.astype(jnp.bfloat16)
            kbh = k_ref[b, h]
            vbh = v_ref[b, h]
            s = lax.dot_general(
                qbh, kbh, (((1,), (1,)), ((), ())),
                preferred_element_type=jnp.float32,
            )
            s = s * 0.125 + mask_add
            m = jnp.max(s, axis=1, keepdims=True)
            w = jnp.exp(s - m)
            w = w / jnp.sum(w, axis=1, keepdims=True)
            cbh = lax.dot_general(
                w.astype(jnp.bfloat16), vbh, (((1,), (0,)), ((), ())),
                preferred_element_type=jnp.float32,
            )
            head_cols.append(cbh)
        ctx_rows.append(jnp.concatenate(head_cols, axis=1))
    ctx = jnp.concatenate(ctx_rows, axis=0)

    partial = lax.dot_general(
        ctx.astype(jnp.bfloat16), wo_ref[...], (((1,), (0,)), ((), ())),
        preferred_element_type=jnp.float32,
    ).reshape(B, SQ, D_MODEL)

    for j in range(N_DEV):
        send_buf[j] = partial[:, j * CH:(j + 1) * CH, :].astype(jnp.bfloat16)

    for t in range(1, N_DEV):
        j = lax.rem(me + t, N_DEV)
        rdma = pltpu.make_async_remote_copy(
            src_ref=send_buf.at[j],
            dst_ref=recv_buf.at[me],
            send_sem=p1_send_sems.at[t],
            recv_sem=p1_recv_sems.at[me],
            device_id=(j,), device_id_type=pl.DeviceIdType.MESH,
        )
        rdma.start()

    recv_buf[me] = send_buf[me]

    for t in range(1, N_DEV):
        s = lax.rem(me - t + N_DEV, N_DEV)
        rdma = pltpu.make_async_remote_copy(
            src_ref=send_buf.at[s],
            dst_ref=recv_buf.at[s],
            send_sem=p1_send_sems.at[t],
            recv_sem=p1_recv_sems.at[s],
            device_id=(s,), device_id_type=pl.DeviceIdType.MESH,
        )
        rdma.wait_recv()

    acc = [recv_buf[s].astype(jnp.float32) for s in range(N_DEV)]
    while len(acc) > 1:
        acc = [acc[i] + acc[i + 1] for i in range(0, len(acc) - 1, 2)] + (
            [acc[-1]] if len(acc) % 2 else []
        )
    red = acc[0].astype(jnp.bfloat16)
    red_buf[...] = red
    out_ref[:, pl.ds(me * CH, CH), :] = red

    for t in range(1, N_DEV):
        j = lax.rem(me + t, N_DEV)
        rdma = pltpu.make_async_remote_copy(
            src_ref=red_buf,
            dst_ref=out_ref.at[:, pl.ds(me * CH, CH), :],
            send_sem=p2_send_sems.at[t],
            recv_sem=p2_recv_sems.at[me],
            device_id=(j,), device_id_type=pl.DeviceIdType.MESH,
        )
        rdma.start()

    for t in range(1, N_DEV):
        s = lax.rem(me - t + N_DEV, N_DEV)
        rdma = pltpu.make_async_remote_copy(
            src_ref=red_buf,
            dst_ref=out_ref.at[:, pl.ds(s * CH, CH), :],
            send_sem=p2_send_sems.at[t],
            recv_sem=p2_recv_sems.at[s],
            device_id=(s,), device_id_type=pl.DeviceIdType.MESH,
        )
        rdma.wait_recv()

    for t in range(1, N_DEV):
        j = lax.rem(me + t, N_DEV)
        s1 = pltpu.make_async_remote_copy(
            src_ref=send_buf.at[j],
            dst_ref=recv_buf.at[me],
            send_sem=p1_send_sems.at[t],
            recv_sem=p1_recv_sems.at[me],
            device_id=(j,), device_id_type=pl.DeviceIdType.MESH,
        )
        s1.wait_send()
        s2 = pltpu.make_async_remote_copy(
            src_ref=red_buf,
            dst_ref=out_ref.at[:, pl.ds(me * CH, CH), :],
            send_sem=p2_send_sems.at[t],
            recv_sem=p2_recv_sems.at[me],
            device_id=(j,), device_id_type=pl.DeviceIdType.MESH,
        )
        s2.wait_send()


def kernel(x, Wq, K_ext, V_ext, Wo):
    me = lax.axis_index("i")
    k = lax.dynamic_slice_in_dim(K_ext, me * H_LOC, H_LOC, axis=2)
    v = lax.dynamic_slice_in_dim(V_ext, me * H_LOC, H_LOC, axis=2)
    k = jnp.transpose(k, (0, 2, 1, 3)).astype(jnp.bfloat16)
    v = jnp.transpose(v, (0, 2, 1, 3)).astype(jnp.bfloat16)

    return pl.pallas_call(
        _body,
        out_shape=jax.ShapeDtypeStruct((B, SQ, D_MODEL), jnp.bfloat16),
        in_specs=[pl.BlockSpec(memory_space=pltpu.VMEM)] * 5,
        out_specs=pl.BlockSpec(memory_space=pltpu.VMEM),
        scratch_shapes=[
            pltpu.VMEM((N_DEV, B, CH, D_MODEL), jnp.bfloat16),
            pltpu.VMEM((N_DEV, B, CH, D_MODEL), jnp.bfloat16),
            pltpu.VMEM((B, CH, D_MODEL), jnp.bfloat16),
            pltpu.SemaphoreType.DMA((N_DEV,)),
            pltpu.SemaphoreType.DMA((N_DEV,)),
            pltpu.SemaphoreType.DMA((N_DEV,)),
            pltpu.SemaphoreType.DMA((N_DEV,)),
        ],
        compiler_params=pltpu.CompilerParams(collective_id=0),
    )(
        x.astype(jnp.bfloat16),
        Wq.astype(jnp.bfloat16),
        k,
        v,
        Wo.astype(jnp.bfloat16),
    )


# device time: 127843 ns/iter; 2.0090x vs baseline; 1.0088x over previous
import numpy as np
import jax
import jax.numpy as jnp
from jax import lax
from jax.experimental import pallas as pl
from jax.experimental.pallas import tpu as pltpu

N_DEV = 32
B = 2
SQ = 512
SKV = 512
HQ_G = 256
DH = 64
H_LOC = HQ_G // N_DEV
CH = SQ // N_DEV
SR = SKV // N_DEV
D_MODEL = 768


def _body(x_ref, wq_ref, ks_ref, vs_ref, wo_ref, out_ref,
          kv_send, kv_recv, k_full, v_full, send_buf, recv_buf, red_buf,
          p0_send_sems, p0_recv_sems,
          p1_send_sems, p1_recv_sems, p2_send_sems, p2_recv_sems):
    me = lax.axis_index("i")

    barrier = pltpu.get_barrier_semaphore()
    for nbr in range(N_DEV):
        @pl.when(nbr != me)
        def _(nbr=nbr):
            pl.semaphore_signal(
                barrier, inc=1,
                device_id=(nbr,), device_id_type=pl.DeviceIdType.MESH,
            )
    pl.semaphore_wait(barrier, N_DEV - 1)

    for j in range(N_DEV):
        kv_send[j, 0] = ks_ref[:, :, j * H_LOC:(j + 1) * H_LOC, :].astype(
            jnp.bfloat16)
        kv_send[j, 1] = vs_ref[:, :, j * H_LOC:(j + 1) * H_LOC, :].astype(
            jnp.bfloat16)

    for t in range(1, N_DEV):
        j = lax.rem(me + t, N_DEV)
        rdma = pltpu.make_async_remote_copy(
            src_ref=kv_send.at[j],
            dst_ref=kv_recv.at[me],
            send_sem=p0_send_sems.at[t],
            recv_sem=p0_recv_sems.at[me],
            device_id=(j,), device_id_type=pl.DeviceIdType.MESH,
        )
        rdma.start()
    kv_recv[me] = kv_send[me]

    xb = x_ref[...].reshape(B * SQ, D_MODEL).astype(jnp.bfloat16)
    q = lax.dot_general(
        xb, wq_ref[...], (((1,), (0,)), ((), ())),
        preferred_element_type=jnp.float32,
    )

    qb = lax.broadcasted_iota(jnp.int32, (SQ, SKV), 0) // 64
    kb = lax.broadcasted_iota(jnp.int32, (SQ, SKV), 1) // 64
    keep = (qb == kb) | ((kb % 4) == (qb % 4))
    mask_add = jnp.where(keep, 0.0, -1e9).astype(jnp.float32)

    for t in range(1, N_DEV):
        s = lax.rem(me - t + N_DEV, N_DEV)
        rdma = pltpu.make_async_remote_copy(
            src_ref=kv_send.at[s],
            dst_ref=kv_recv.at[s],
            send_sem=p0_send_sems.at[t],
            recv_sem=p0_recv_sems.at[s],
            device_id=(s,), device_id_type=pl.DeviceIdType.MESH,
        )
        rdma.wait_recv()

    for s in range(N_DEV):
        k_full[:, s * SR:(s + 1) * SR, :, :] = kv_recv[s, 0]
        v_full[:, s * SR:(s + 1) * SR, :, :] = kv_recv[s, 1]

    ctx_rows = []
    for b in range(B):
        head_cols = []
        for h in range(H_LOC):
            qbh = q[b * SQ:(b + 1) * SQ, h * DH:(h + 1) * DH]
            qbh = qbh.astype(jnp.bfloat16)
            kbh = k_full[b, :, h, :]
            vbh = v_full[b, :, h, :]
            s = lax.dot_general(
                qbh, kbh, (((1,), (1,)), ((), ())),
                preferred_element_type=jnp.float32,
            )
            s = s * 0.125 + mask_add
            m = jnp.max(s, axis=1, keepdims=True)
            w = jnp.exp(s - m)
            w = w / jnp.sum(w, axis=1, keepdims=True)
            cbh = lax.dot_general(
                w.astype(jnp.bfloat16), vbh, (((1,), (0,)), ((), ())),
                preferred_element_type=jnp.float32,
            )
            head_cols.append(cbh)
        ctx_rows.append(jnp.concatenate(head_cols, axis=1))
    ctx = jnp.concatenate(ctx_rows, axis=0)

    partial = lax.dot_general(
        ctx.astype(jnp.bfloat16), wo_ref[...], (((1,), (0,)), ((), ())),
        preferred_element_type=jnp.float32,
    ).reshape(B, SQ, D_MODEL)

    for j in range(N_DEV):
        send_buf[j] = partial[:, j * CH:(j + 1) * CH, :].astype(jnp.bfloat16)

    for t in range(1, N_DEV):
        j = lax.rem(me + t, N_DEV)
        rdma = pltpu.make_async_remote_copy(
            src_ref=send_buf.at[j],
            dst_ref=recv_buf.at[me],
            send_sem=p1_send_sems.at[t],
            recv_sem=p1_recv_sems.at[me],
            device_id=(j,), device_id_type=pl.DeviceIdType.MESH,
        )
        rdma.start()

    recv_buf[me] = send_buf[me]

    for t in range(1, N_DEV):
        s = lax.rem(me - t + N_DEV, N_DEV)
        rdma = pltpu.make_async_remote_copy(
            src_ref=send_buf.at[s],
            dst_ref=recv_buf.at[s],
            send_sem=p1_send_sems.at[t],
            recv_sem=p1_recv_sems.at[s],
            device_id=(s,), device_id_type=pl.DeviceIdType.MESH,
        )
        rdma.wait_recv()

    acc = [recv_buf[s].astype(jnp.float32) for s in range(N_DEV)]
    while len(acc) > 1:
        acc = [acc[i] + acc[i + 1] for i in range(0, len(acc) - 1, 2)] + (
            [acc[-1]] if len(acc) % 2 else []
        )
    red = acc[0].astype(jnp.bfloat16)
    red_buf[...] = red
    out_ref[:, pl.ds(me * CH, CH), :] = red

    for t in range(1, N_DEV):
        j = lax.rem(me + t, N_DEV)
        rdma = pltpu.make_async_remote_copy(
            src_ref=red_buf,
            dst_ref=out_ref.at[:, pl.ds(me * CH, CH), :],
            send_sem=p2_send_sems.at[t],
            recv_sem=p2_recv_sems.at[me],
            device_id=(j,), device_id_type=pl.DeviceIdType.MESH,
        )
        rdma.start()

    for t in range(1, N_DEV):
        s = lax.rem(me - t + N_DEV, N_DEV)
        rdma = pltpu.make_async_remote_copy(
            src_ref=red_buf,
            dst_ref=out_ref.at[:, pl.ds(s * CH, CH), :],
            send_sem=p2_send_sems.at[t],
            recv_sem=p2_recv_sems.at[s],
            device_id=(s,), device_id_type=pl.DeviceIdType.MESH,
        )
        rdma.wait_recv()

    for t in range(1, N_DEV):
        j = lax.rem(me + t, N_DEV)
        s0 = pltpu.make_async_remote_copy(
            src_ref=kv_send.at[j],
            dst_ref=kv_recv.at[me],
            send_sem=p0_send_sems.at[t],
            recv_sem=p0_recv_sems.at[me],
            device_id=(j,), device_id_type=pl.DeviceIdType.MESH,
        )
        s0.wait_send()
        s1 = pltpu.make_async_remote_copy(
            src_ref=send_buf.at[j],
            dst_ref=recv_buf.at[me],
            send_sem=p1_send_sems.at[t],
            recv_sem=p1_recv_sems.at[me],
            device_id=(j,), device_id_type=pl.DeviceIdType.MESH,
        )
        s1.wait_send()
        s2 = pltpu.make_async_remote_copy(
            src_ref=red_buf,
            dst_ref=out_ref.at[:, pl.ds(me * CH, CH), :],
            send_sem=p2_send_sems.at[t],
            recv_sem=p2_recv_sems.at[me],
            device_id=(j,), device_id_type=pl.DeviceIdType.MESH,
        )
        s2.wait_send()


def kernel(x, Wq, K_ext, V_ext, Wo):
    me = lax.axis_index("i")
    ks = lax.dynamic_slice_in_dim(K_ext, me * SR, SR, axis=1)
    vs = lax.dynamic_slice_in_dim(V_ext, me * SR, SR, axis=1)

    return pl.pallas_call(
        _body,
        out_shape=jax.ShapeDtypeStruct((B, SQ, D_MODEL), jnp.bfloat16),
        in_specs=[pl.BlockSpec(memory_space=pltpu.VMEM)] * 5,
        out_specs=pl.BlockSpec(memory_space=pltpu.VMEM),
        scratch_shapes=[
            pltpu.VMEM((N_DEV, 2, B, SR, H_LOC, DH), jnp.bfloat16),
            pltpu.VMEM((N_DEV, 2, B, SR, H_LOC, DH), jnp.bfloat16),
            pltpu.VMEM((B, SKV, H_LOC, DH), jnp.bfloat16),
            pltpu.VMEM((B, SKV, H_LOC, DH), jnp.bfloat16),
            pltpu.VMEM((N_DEV, B, CH, D_MODEL), jnp.bfloat16),
            pltpu.VMEM((N_DEV, B, CH, D_MODEL), jnp.bfloat16),
            pltpu.VMEM((B, CH, D_MODEL), jnp.bfloat16),
            pltpu.SemaphoreType.DMA((N_DEV,)),
            pltpu.SemaphoreType.DMA((N_DEV,)),
            pltpu.SemaphoreType.DMA((N_DEV,)),
            pltpu.SemaphoreType.DMA((N_DEV,)),
            pltpu.SemaphoreType.DMA((N_DEV,)),
            pltpu.SemaphoreType.DMA((N_DEV,)),
        ],
        compiler_params=pltpu.CompilerParams(collective_id=0),
    )(
        x.astype(jnp.bfloat16),
        Wq.astype(jnp.bfloat16),
        ks,
        vs,
        Wo.astype(jnp.bfloat16),
    )


# device time: 127385 ns/iter; 2.0162x vs baseline; 1.0036x over previous
import numpy as np
import jax
import jax.numpy as jnp
from jax import lax
from jax.experimental import pallas as pl
from jax.experimental.pallas import tpu as pltpu

N_DEV = 32
B = 2
SQ = 512
SKV = 512
HQ_G = 256
DH = 64
H_LOC = HQ_G // N_DEV
CH = SQ // N_DEV
SR = SKV // N_DEV
HH = 2 * H_LOC
D_MODEL = 768


def _body(x_ref, wq_ref, ks_ref, vs_ref, wo_ref, out_ref,
          kv_all, kv_full, send_buf, recv_buf, red_buf,
          own_sem,
          p0_send_sems, p0_recv_sems,
          p1_send_sems, p1_recv_sems, p2_send_sems, p2_recv_sems):
    me = lax.axis_index("i")

    barrier = pltpu.get_barrier_semaphore()
    for nbr in range(N_DEV):
        @pl.when(nbr != me)
        def _(nbr=nbr):
            pl.semaphore_signal(
                barrier, inc=1,
                device_id=(nbr,), device_id_type=pl.DeviceIdType.MESH,
            )
    pl.semaphore_wait(barrier, N_DEV - 1)

    for j in range(N_DEV):
        kv_all[:, :, j * HH:j * HH + H_LOC, :] = (
            ks_ref[:, :, j * H_LOC:(j + 1) * H_LOC, :].astype(jnp.bfloat16))
        kv_all[:, :, j * HH + H_LOC:(j + 1) * HH, :] = (
            vs_ref[:, :, j * H_LOC:(j + 1) * H_LOC, :].astype(jnp.bfloat16))

    for t in range(1, N_DEV):
        j = lax.rem(me + t, N_DEV)
        rdma = pltpu.make_async_remote_copy(
            src_ref=kv_all.at[:, :, pl.ds(j * HH, HH), :],
            dst_ref=kv_full.at[:, pl.ds(me * SR, SR), :, :],
            send_sem=p0_send_sems.at[t],
            recv_sem=p0_recv_sems.at[me],
            device_id=(j,), device_id_type=pl.DeviceIdType.MESH,
        )
        rdma.start()

    own = pltpu.make_async_copy(
        kv_all.at[:, :, pl.ds(me * HH, HH), :],
        kv_full.at[:, pl.ds(me * SR, SR), :, :],
        own_sem,
    )
    own.start()

    xb = x_ref[...].reshape(B * SQ, D_MODEL).astype(jnp.bfloat16)
    q = lax.dot_general(
        xb, wq_ref[...], (((1,), (0,)), ((), ())),
        preferred_element_type=jnp.float32,
    )

    qb = lax.broadcasted_iota(jnp.int32, (SQ, SKV), 0) // 64
    kb = lax.broadcasted_iota(jnp.int32, (SQ, SKV), 1) // 64
    keep = (qb == kb) | ((kb % 4) == (qb % 4))
    mask_add = jnp.where(keep, 0.0, -1e9).astype(jnp.float32)

    own.wait()
    for t in range(1, N_DEV):
        s = lax.rem(me - t + N_DEV, N_DEV)
        rdma = pltpu.make_async_remote_copy(
            src_ref=kv_all.at[:, :, pl.ds(s * HH, HH), :],
            dst_ref=kv_full.at[:, pl.ds(s * SR, SR), :, :],
            send_sem=p0_send_sems.at[t],
            recv_sem=p0_recv_sems.at[s],
            device_id=(s,), device_id_type=pl.DeviceIdType.MESH,
        )
        rdma.wait_recv()

    ctx_rows = []
    for b in range(B):
        head_cols = []
        for h in range(H_LOC):
            qbh = q[b * SQ:(b + 1) * SQ, h * DH:(h + 1) * DH]
            qbh = qbh.astype(jnp.bfloat16)
            kbh = kv_full[b, :, h, :]
            vbh = kv_full[b, :, H_LOC + h, :]
            s = lax.dot_general(
                qbh, kbh, (((1,), (1,)), ((), ())),
                preferred_element_type=jnp.float32,
            )
            s = s * 0.125 + mask_add
            m = jnp.max(s, axis=1, keepdims=True)
            w = jnp.exp(s - m)
            w = w / jnp.sum(w, axis=1, keepdims=True)
            cbh = lax.dot_general(
                w.astype(jnp.bfloat16), vbh, (((1,), (0,)), ((), ())),
                preferred_element_type=jnp.float32,
            )
            head_cols.append(cbh)
        ctx_rows.append(jnp.concatenate(head_cols, axis=1))
    ctx = jnp.concatenate(ctx_rows, axis=0)

    partial = lax.dot_general(
        ctx.astype(jnp.bfloat16), wo_ref[...], (((1,), (0,)), ((), ())),
        preferred_element_type=jnp.float32,
    ).reshape(B, SQ, D_MODEL)

    for j in range(N_DEV):
        send_buf[j] = partial[:, j * CH:(j + 1) * CH, :].astype(jnp.bfloat16)

    for t in range(1, N_DEV):
        j = lax.rem(me + t, N_DEV)
        rdma = pltpu.make_async_remote_copy(
            src_ref=send_buf.at[j],
            dst_ref=recv_buf.at[me],
            send_sem=p1_send_sems.at[t],
            recv_sem=p1_recv_sems.at[me],
            device_id=(j,), device_id_type=pl.DeviceIdType.MESH,
        )
        rdma.start()

    recv_buf[me] = send_buf[me]

    for t in range(1, N_DEV):
        s = lax.rem(me - t + N_DEV, N_DEV)
        rdma = pltpu.make_async_remote_copy(
            src_ref=send_buf.at[s],
            dst_ref=recv_buf.at[s],
            send_sem=p1_send_sems.at[t],
            recv_sem=p1_recv_sems.at[s],
            device_id=(s,), device_id_type=pl.DeviceIdType.MESH,
        )
        rdma.wait_recv()

    acc = [recv_buf[s].astype(jnp.float32) for s in range(N_DEV)]
    while len(acc) > 1:
        acc = [acc[i] + acc[i + 1] for i in range(0, len(acc) - 1, 2)] + (
            [acc[-1]] if len(acc) % 2 else []
        )
    red = acc[0].astype(jnp.bfloat16)
    red_buf[...] = red
    out_ref[:, pl.ds(me * CH, CH), :] = red

    for t in range(1, N_DEV):
        j = lax.rem(me + t, N_DEV)
        rdma = pltpu.make_async_remote_copy(
            src_ref=red_buf,
            dst_ref=out_ref.at[:, pl.ds(me * CH, CH), :],
            send_sem=p2_send_sems.at[t],
            recv_sem=p2_recv_sems.at[me],
            device_id=(j,), device_id_type=pl.DeviceIdType.MESH,
        )
        rdma.start()

    for t in range(1, N_DEV):
        s = lax.rem(me - t + N_DEV, N_DEV)
        rdma = pltpu.make_async_remote_copy(
            src_ref=red_buf,
            dst_ref=out_ref.at[:, pl.ds(s * CH, CH), :],
            send_sem=p2_send_sems.at[t],
            recv_sem=p2_recv_sems.at[s],
            device_id=(s,), device_id_type=pl.DeviceIdType.MESH,
        )
        rdma.wait_recv()

    for t in range(1, N_DEV):
        j = lax.rem(me + t, N_DEV)
        s0 = pltpu.make_async_remote_copy(
            src_ref=kv_all.at[:, :, pl.ds(j * HH, HH), :],
            dst_ref=kv_full.at[:, pl.ds(me * SR, SR), :, :],
            send_sem=p0_send_sems.at[t],
            recv_sem=p0_recv_sems.at[me],
            device_id=(j,), device_id_type=pl.DeviceIdType.MESH,
        )
        s0.wait_send()
        s1 = pltpu.make_async_remote_copy(
            src_ref=send_buf.at[j],
            dst_ref=recv_buf.at[me],
            send_sem=p1_send_sems.at[t],
            recv_sem=p1_recv_sems.at[me],
            device_id=(j,), device_id_type=pl.DeviceIdType.MESH,
        )
        s1.wait_send()
        s2 = pltpu.make_async_remote_copy(
            src_ref=red_buf,
            dst_ref=out_ref.at[:, pl.ds(me * CH, CH), :],
            send_sem=p2_send_sems.at[t],
            recv_sem=p2_recv_sems.at[me],
            device_id=(j,), device_id_type=pl.DeviceIdType.MESH,
        )
        s2.wait_send()


def kernel(x, Wq, K_ext, V_ext, Wo):
    me = lax.axis_index("i")
    ks = lax.dynamic_slice_in_dim(K_ext, me * SR, SR, axis=1)
    vs = lax.dynamic_slice_in_dim(V_ext, me * SR, SR, axis=1)

    return pl.pallas_call(
        _body,
        out_shape=jax.ShapeDtypeStruct((B, SQ, D_MODEL), jnp.bfloat16),
        in_specs=[pl.BlockSpec(memory_space=pltpu.VMEM)] * 5,
        out_specs=pl.BlockSpec(memory_space=pltpu.VMEM),
        scratch_shapes=[
            pltpu.VMEM((B, SR, N_DEV * HH, DH), jnp.bfloat16),
            pltpu.VMEM((B, SKV, HH, DH), jnp.bfloat16),
            pltpu.VMEM((N_DEV, B, CH, D_MODEL), jnp.bfloat16),
            pltpu.VMEM((N_DEV, B, CH, D_MODEL), jnp.bfloat16),
            pltpu.VMEM((B, CH, D_MODEL), jnp.bfloat16),
            pltpu.SemaphoreType.DMA,
            pltpu.SemaphoreType.DMA((N_DEV,)),
            pltpu.SemaphoreType.DMA((N_DEV,)),
            pltpu.SemaphoreType.DMA((N_DEV,)),
            pltpu.SemaphoreType.DMA((N_DEV,)),
            pltpu.SemaphoreType.DMA((N_DEV,)),
            pltpu.SemaphoreType.DMA((N_DEV,)),
        ],
        compiler_params=pltpu.CompilerParams(collective_id=0),
    )(
        x.astype(jnp.bfloat16),
        Wq.astype(jnp.bfloat16),
        ks,
        vs,
        Wo.astype(jnp.bfloat16),
    )


# device time: 126844 ns/iter; 2.0248x vs baseline; 1.0043x over previous
import numpy as np
import jax
import jax.numpy as jnp
from jax import lax
from jax.experimental import pallas as pl
from jax.experimental.pallas import tpu as pltpu

N_DEV = 32
B = 2
SQ = 512
SKV = 512
HQ_G = 256
DH = 64
H_LOC = HQ_G // N_DEV
CH = SQ // N_DEV
SR = SKV // N_DEV
HH = 2 * H_LOC
D_MODEL = 768


def _body(x_ref, wq_ref, ks_ref, vs_ref, wo_ref, out_ref,
          kv_all, kv_full, send_buf, recv_buf, red_buf,
          own_sem,
          p0_send_sems, p0_recv_sems,
          p1_send_sems, p1_recv_sems, p2_send_sems, p2_recv_sems):
    me = lax.axis_index("i")

    barrier = pltpu.get_barrier_semaphore()
    for nbr in range(N_DEV):
        @pl.when(nbr != me)
        def _(nbr=nbr):
            pl.semaphore_signal(
                barrier, inc=1,
                device_id=(nbr,), device_id_type=pl.DeviceIdType.MESH,
            )
    pl.semaphore_wait(barrier, N_DEV - 1)

    for j in range(N_DEV):
        kv_all[j, :, :, 0:H_LOC, :] = (
            ks_ref[:, :, j * H_LOC:(j + 1) * H_LOC, :].astype(jnp.bfloat16))
        kv_all[j, :, :, H_LOC:HH, :] = (
            vs_ref[:, :, j * H_LOC:(j + 1) * H_LOC, :].astype(jnp.bfloat16))

    for t in range(1, N_DEV):
        j = lax.rem(me + t, N_DEV)
        rdma = pltpu.make_async_remote_copy(
            src_ref=kv_all.at[j],
            dst_ref=kv_full.at[:, pl.ds(me * SR, SR), :, :],
            send_sem=p0_send_sems.at[t],
            recv_sem=p0_recv_sems.at[me],
            device_id=(j,), device_id_type=pl.DeviceIdType.MESH,
        )
        rdma.start()

    own = pltpu.make_async_copy(
        kv_all.at[me],
        kv_full.at[:, pl.ds(me * SR, SR), :, :],
        own_sem,
    )
    own.start()

    xb = x_ref[...].reshape(B * SQ, D_MODEL).astype(jnp.bfloat16)
    q = lax.dot_general(
        xb, wq_ref[...], (((1,), (0,)), ((), ())),
        preferred_element_type=jnp.float32,
    )

    qb = lax.broadcasted_iota(jnp.int32, (SQ, SKV), 0) // 64
    kb = lax.broadcasted_iota(jnp.int32, (SQ, SKV), 1) // 64
    keep = (qb == kb) | ((kb % 4) == (qb % 4))
    mask_add = jnp.where(keep, 0.0, -1e9).astype(jnp.float32)

    own.wait()
    for t in range(1, N_DEV):
        s = lax.rem(me - t + N_DEV, N_DEV)
        rdma = pltpu.make_async_remote_copy(
            src_ref=kv_all.at[s],
            dst_ref=kv_full.at[:, pl.ds(s * SR, SR), :, :],
            send_sem=p0_send_sems.at[t],
            recv_sem=p0_recv_sems.at[s],
            device_id=(s,), device_id_type=pl.DeviceIdType.MESH,
        )
        rdma.wait_recv()

    ctx_rows = []
    for b in range(B):
        head_cols = []
        for h in range(H_LOC):
            qbh = q[b * SQ:(b + 1) * SQ, h * DH:(h + 1) * DH]
            qbh = qbh.astype(jnp.bfloat16)
            kbh = kv_full[b, :, h, :]
            vbh = kv_full[b, :, H_LOC + h, :]
            s = lax.dot_general(
                qbh, kbh, (((1,), (1,)), ((), ())),
                preferred_element_type=jnp.float32,
            )
            s = s * 0.125 + mask_add
            m = jnp.max(s, axis=1, keepdims=True)
            w = jnp.exp(s - m)
            w = w / jnp.sum(w, axis=1, keepdims=True)
            cbh = lax.dot_general(
                w.astype(jnp.bfloat16), vbh, (((1,), (0,)), ((), ())),
                preferred_element_type=jnp.float32,
            )
            head_cols.append(cbh)
        ctx_rows.append(jnp.concatenate(head_cols, axis=1))
    ctx = jnp.concatenate(ctx_rows, axis=0)

    partial = lax.dot_general(
        ctx.astype(jnp.bfloat16), wo_ref[...], (((1,), (0,)), ((), ())),
        preferred_element_type=jnp.float32,
    ).reshape(B, SQ, D_MODEL)

    for j in range(N_DEV):
        send_buf[j] = partial[:, j * CH:(j + 1) * CH, :].astype(jnp.bfloat16)

    for t in range(1, N_DEV):
        j = lax.rem(me + t, N_DEV)
        rdma = pltpu.make_async_remote_copy(
            src_ref=send_buf.at[j],
            dst_ref=recv_buf.at[me],
            send_sem=p1_send_sems.at[t],
            recv_sem=p1_recv_sems.at[me],
            device_id=(j,), device_id_type=pl.DeviceIdType.MESH,
        )
        rdma.start()

    recv_buf[me] = send_buf[me]

    for t in range(1, N_DEV):
        s = lax.rem(me - t + N_DEV, N_DEV)
        rdma = pltpu.make_async_remote_copy(
            src_ref=send_buf.at[s],
            dst_ref=recv_buf.at[s],
            send_sem=p1_send_sems.at[t],
            recv_sem=p1_recv_sems.at[s],
            device_id=(s,), device_id_type=pl.DeviceIdType.MESH,
        )
        rdma.wait_recv()

    acc = [recv_buf[s].astype(jnp.float32) for s in range(N_DEV)]
    while len(acc) > 1:
        acc = [acc[i] + acc[i + 1] for i in range(0, len(acc) - 1, 2)] + (
            [acc[-1]] if len(acc) % 2 else []
        )
    red = acc[0].astype(jnp.bfloat16)
    red_buf[...] = red
    out_ref[:, pl.ds(me * CH, CH), :] = red

    for t in range(1, N_DEV):
        j = lax.rem(me + t, N_DEV)
        rdma = pltpu.make_async_remote_copy(
            src_ref=red_buf,
            dst_ref=out_ref.at[:, pl.ds(me * CH, CH), :],
            send_sem=p2_send_sems.at[t],
            recv_sem=p2_recv_sems.at[me],
            device_id=(j,), device_id_type=pl.DeviceIdType.MESH,
        )
        rdma.start()

    for t in range(1, N_DEV):
        s = lax.rem(me - t + N_DEV, N_DEV)
        rdma = pltpu.make_async_remote_copy(
            src_ref=red_buf,
            dst_ref=out_ref.at[:, pl.ds(s * CH, CH), :],
            send_sem=p2_send_sems.at[t],
            recv_sem=p2_recv_sems.at[s],
            device_id=(s,), device_id_type=pl.DeviceIdType.MESH,
        )
        rdma.wait_recv()

    for t in range(1, N_DEV):
        j = lax.rem(me + t, N_DEV)
        s0 = pltpu.make_async_remote_copy(
            src_ref=kv_all.at[j],
            dst_ref=kv_full.at[:, pl.ds(me * SR, SR), :, :],
            send_sem=p0_send_sems.at[t],
            recv_sem=p0_recv_sems.at[me],
            device_id=(j,), device_id_type=pl.DeviceIdType.MESH,
        )
        s0.wait_send()
        s1 = pltpu.make_async_remote_copy(
            src_ref=send_buf.at[j],
            dst_ref=recv_buf.at[me],
            send_sem=p1_send_sems.at[t],
            recv_sem=p1_recv_sems.at[me],
            device_id=(j,), device_id_type=pl.DeviceIdType.MESH,
        )
        s1.wait_send()
        s2 = pltpu.make_async_remote_copy(
            src_ref=red_buf,
            dst_ref=out_ref.at[:, pl.ds(me * CH, CH), :],
            send_sem=p2_send_sems.at[t],
            recv_sem=p2_recv_sems.at[me],
            device_id=(j,), device_id_type=pl.DeviceIdType.MESH,
        )
        s2.wait_send()


def kernel(x, Wq, K_ext, V_ext, Wo):
    me = lax.axis_index("i")
    ks = lax.dynamic_slice_in_dim(K_ext, me * SR, SR, axis=1)
    vs = lax.dynamic_slice_in_dim(V_ext, me * SR, SR, axis=1)

    return pl.pallas_call(
        _body,
        out_shape=jax.ShapeDtypeStruct((B, SQ, D_MODEL), jnp.bfloat16),
        in_specs=[pl.BlockSpec(memory_space=pltpu.VMEM)] * 5,
        out_specs=pl.BlockSpec(memory_space=pltpu.VMEM),
        scratch_shapes=[
            pltpu.VMEM((N_DEV, B, SR, HH, DH), jnp.bfloat16),
            pltpu.VMEM((B, SKV, HH, DH), jnp.bfloat16),
            pltpu.VMEM((N_DEV, B, CH, D_MODEL), jnp.bfloat16),
            pltpu.VMEM((N_DEV, B, CH, D_MODEL), jnp.bfloat16),
            pltpu.VMEM((B, CH, D_MODEL), jnp.bfloat16),
            pltpu.SemaphoreType.DMA,
            pltpu.SemaphoreType.DMA((N_DEV,)),
            pltpu.SemaphoreType.DMA((N_DEV,)),
            pltpu.SemaphoreType.DMA((N_DEV,)),
            pltpu.SemaphoreType.DMA((N_DEV,)),
            pltpu.SemaphoreType.DMA((N_DEV,)),
            pltpu.SemaphoreType.DMA((N_DEV,)),
        ],
        compiler_params=pltpu.CompilerParams(collective_id=0),
    )(
        x.astype(jnp.bfloat16),
        Wq.astype(jnp.bfloat16),
        ks,
        vs,
        Wo.astype(jnp.bfloat16),
    )
